# baseline (speedup 1.0000x reference)
import functools

import jax
import jax.numpy as jnp
from jax import lax
from jax.experimental import pallas as pl
from jax.experimental.pallas import tpu as pltpu

TOPIC = 1024
HID = 1024
K = 128
L = 4096

_INT_MIN = -2147483648


def _attn_body(v_ref, h_ref, ws_ref, b_ref, vs_ref, hs_ref, score_ref):
    vrow = v_ref[...]
    alpha = lax.dot_general(
        vrow, vs_ref[...], (((1,), (1,)), ((), ())),
        preferred_element_type=jnp.float32)

    m = jnp.max(alpha)
    ybits = lax.bitcast_convert_type(alpha, jnp.int32)
    imin = jnp.int32(_INT_MIN)
    mono = jnp.where(ybits >= 0, ybits,
                     jnp.bitwise_not(jnp.bitwise_xor(ybits, imin)))

    def bit_step(i, tu):
        bit = jnp.left_shift(jnp.int32(1), 31 - i)
        tc = jnp.bitwise_or(tu, bit)
        ts = jnp.bitwise_xor(tc, _INT_MIN)
        cnt = jnp.sum((mono >= ts).astype(jnp.int32))
        return jnp.where(cnt >= K, tc, tu)

    tu = lax.fori_loop(0, 32, bit_step, jnp.int32(0))
    thr = jnp.bitwise_xor(tu, _INT_MIN)

    gt = mono > thr
    eq = mono == thr
    need = K - jnp.sum(gt.astype(jnp.int32))
    iota = lax.broadcasted_iota(jnp.int32, (1, L), 1)

    def cbit_step(i, c):
        bit = jnp.left_shift(jnp.int32(1), 12 - i)
        cc = jnp.bitwise_or(c, bit)
        cnt = jnp.sum((eq & (iota < cc)).astype(jnp.int32))
        return jnp.where(cnt <= need, cc, c)

    c = lax.fori_loop(0, 13, cbit_step, jnp.int32(0))
    sel = gt | (eq & (iota < c))

    e = jnp.where(sel, jnp.exp(alpha - m), 0.0)
    w = e / jnp.sum(e)
    attn = jnp.dot(w, hs_ref[...],
                   preferred_element_type=jnp.float32)

    hrow = h_ref[...]
    sc = (jnp.sum(vrow * ws_ref[:, 0:TOPIC])
          + jnp.sum(attn * ws_ref[:, TOPIC:TOPIC + HID])
          + jnp.sum(hrow * ws_ref[:, TOPIC + HID:TOPIC + 2 * HID])
          + float(K) * ws_ref[0, TOPIC + 2 * HID]
          + b_ref[0, 0])
    score_ref[...] = jnp.broadcast_to(sc, (1, 1))


def _gru_body(pos_ref, v_ref, s_ref, h_ref, wl_ref, bih_ref, bhh_ref,
              wab_ref, whh_ref, out_ref):
    del pos_ref
    vrow = v_ref[...]
    hrow = h_ref[...]
    gi = (lax.dot_general(vrow, wab_ref[...], (((1,), (1,)), ((), ())),
                          preferred_element_type=jnp.float32)
          + s_ref[0, 0] * wl_ref[...] + bih_ref[...])
    gh = (lax.dot_general(hrow, whh_ref[...], (((1,), (1,)), ((), ())),
                          preferred_element_type=jnp.float32)
          + bhh_ref[...])
    r = jax.nn.sigmoid(gi[:, 0:HID] + gh[:, 0:HID])
    z = jax.nn.sigmoid(gi[:, HID:2 * HID] + gh[:, HID:2 * HID])
    n = jnp.tanh(gi[:, 2 * HID:] + r * gh[:, 2 * HID:])
    out_ref[...] = (1.0 - z) * n + z * hrow


def kernel(v, s, h, vs, hs, W_ih, W_hh, b_ih, b_hh, W_score, b_score):
    vrow = v.reshape(1, TOPIC)
    hrow = h.reshape(1, HID)
    s11 = s.reshape(1, 1)

    score = pl.pallas_call(
        _attn_body,
        out_shape=jax.ShapeDtypeStruct((1, 1), jnp.float32),
        in_specs=[
            pl.BlockSpec((1, TOPIC), lambda: (0, 0)),
            pl.BlockSpec((1, HID), lambda: (0, 0)),
            pl.BlockSpec((1, TOPIC + 2 * HID + 1), lambda: (0, 0)),
            pl.BlockSpec((1, 1), lambda: (0, 0)),
            pl.BlockSpec((L, TOPIC), lambda: (0, 0)),
            pl.BlockSpec((L, HID), lambda: (0, 0)),
        ],
        out_specs=pl.BlockSpec((1, 1), lambda: (0, 0)),
    )(vrow, hrow, W_score, b_score.reshape(1, 1), vs, hs)

    pos = (s >= 0.5).astype(jnp.int32)
    W_ab = W_ih[:, :2 * TOPIC]
    w_last = W_ih[:, 2 * TOPIC].reshape(1, 3 * HID)
    bih_row = b_ih.reshape(1, 3 * HID)
    bhh_row = b_hh.reshape(1, 3 * HID)

    grid_spec = pltpu.PrefetchScalarGridSpec(
        num_scalar_prefetch=1,
        grid=(1,),
        in_specs=[
            pl.BlockSpec((1, TOPIC), lambda i, p: (0, 0)),
            pl.BlockSpec((1, 1), lambda i, p: (0, 0)),
            pl.BlockSpec((1, HID), lambda i, p: (0, 0)),
            pl.BlockSpec((1, 3 * HID), lambda i, p: (0, 0)),
            pl.BlockSpec((1, 3 * HID), lambda i, p: (0, 0)),
            pl.BlockSpec((1, 3 * HID), lambda i, p: (0, 0)),
            pl.BlockSpec((3 * HID, TOPIC), lambda i, p: (0, 1 - p[0])),
            pl.BlockSpec((3 * HID, HID), lambda i, p: (0, 0)),
        ],
        out_specs=pl.BlockSpec((1, HID), lambda i, p: (0, 0)),
    )
    h_new = pl.pallas_call(
        _gru_body,
        grid_spec=grid_spec,
        out_shape=jax.ShapeDtypeStruct((1, HID), jnp.float32),
    )(pos, vrow, s11, hrow, w_last, bih_row, bhh_row, W_ab, W_hh)

    return (score, h_new.reshape(1, 1, HID))
